# Initial kernel scaffold; baseline (speedup 1.0000x reference)
import jax
import jax.numpy as jnp
from jax.experimental import pallas as pl


def kernel(features, edge_src, edge_dst):
    raise NotImplementedError("write your pallas kernel here")


# speedup vs baseline: 3.1309x; 3.1309x over previous
import functools

import jax
import jax.numpy as jnp
from jax import lax
from jax.experimental import pallas as pl
from jax.experimental.pallas import tpu as pltpu
from jax.experimental.pallas import tpu_sc as plsc

NC = 2
NS = 16
NW = NC * NS
EPW = 128
IDXB = 8
R_CHUNKS = 80
R = R_CHUNKS * 128


def _zero_rows(rows_v):
    zf = jnp.zeros((16,), jnp.float32)

    @pl.loop(0, EPW)
    def _init(i):
        for k in range(8):
            rows_v[i, pl.ds(k * 16, 16)] = zf


def _sums_body(feat, srcp, dstp, sums_out,
               sh_sums, src_v, dst_v, rows_v, sem):
    c = lax.axis_index("c")
    s = lax.axis_index("s")
    w = c * NS + s
    rows_per_w = srcp.shape[0] // NW

    _zero_rows(rows_v)

    for k in range(R_CHUNKS // NS):
        chunk = k * NS + s
        pltpu.sync_copy(rows_v, sh_sums.at[pl.ds(chunk * 128, 128)])

    plsc.subcore_barrier()

    @pl.loop(0, rows_per_w // IDXB)
    def _main(b):
        base = w * rows_per_w + b * IDXB
        pltpu.sync_copy(srcp.at[pl.ds(base, IDXB)], src_v)
        pltpu.sync_copy(dstp.at[pl.ds(base, IDXB)], dst_v)
        for jj in range(IDXB):
            pltpu.async_copy(feat.at[src_v.at[jj]], rows_v, sem).wait()
            pltpu.sync_copy(rows_v, sh_sums.at[dst_v.at[jj]], add=True)

    plsc.subcore_barrier()

    for k in range(R_CHUNKS // NS):
        chunk = k * NS + s
        pltpu.sync_copy(sh_sums.at[pl.ds(chunk * 128, 128)], rows_v)
        pltpu.sync_copy(rows_v, sums_out.at[pl.ds(c * R + chunk * 128, 128)])


def _cnts_body(dstp, cnts_out, sh_cnts, dst_v, ones_v, sem):
    c = lax.axis_index("c")
    s = lax.axis_index("s")
    w = c * NS + s
    rows_per_w = dstp.shape[0] // NW
    of = jnp.ones((16,), jnp.float32)

    _zero_rows(ones_v)

    for k in range(R_CHUNKS // NS):
        chunk = k * NS + s
        pltpu.sync_copy(ones_v, sh_cnts.at[pl.ds(chunk * 128, 128)])

    @pl.loop(0, EPW)
    def _setones(i):
        for k in range(8):
            ones_v[i, pl.ds(k * 16, 16)] = of

    plsc.subcore_barrier()

    @pl.loop(0, rows_per_w // IDXB)
    def _main(b):
        base = w * rows_per_w + b * IDXB
        pltpu.sync_copy(dstp.at[pl.ds(base, IDXB)], dst_v)
        for jj in range(IDXB):
            pltpu.sync_copy(ones_v, sh_cnts.at[dst_v.at[jj]], add=True)

    plsc.subcore_barrier()

    for k in range(R_CHUNKS // NS):
        chunk = k * NS + s
        pltpu.sync_copy(sh_cnts.at[pl.ds(chunk * 128, 128)], ones_v)
        pltpu.sync_copy(ones_v, cnts_out.at[pl.ds(c * R + chunk * 128, 128)])


def _combine_body(s0, s1, c0, c1, o):
    cnt = c0[:, 0:1] + c1[:, 0:1]
    o[:, :] = (s0[:, :] + s1[:, :]) / jnp.maximum(cnt, 1.0)


def kernel(features, edge_src, edge_dst):
    n_nodes, d = features.shape
    e = edge_src.shape[0]
    rows_per_w = -(-e // (NW * EPW))
    rows_per_w = -(-rows_per_w // IDXB) * IDXB
    e_pad = NW * EPW * rows_per_w
    pad = e_pad - e

    srcp = jnp.concatenate(
        [edge_src, jnp.zeros((pad,), jnp.int32)]).reshape(-1, EPW)
    dstp = jnp.concatenate(
        [edge_dst, jnp.full((pad,), n_nodes, jnp.int32)]).reshape(-1, EPW)

    mesh = plsc.VectorSubcoreMesh(
        core_axis_name="c", subcore_axis_name="s",
        num_cores=NC, num_subcores=NS)

    sums_call = pl.kernel(
        _sums_body,
        out_type=jax.ShapeDtypeStruct((NC * R, d), jnp.float32),
        mesh=mesh,
        scratch_types=[
            pltpu.VMEM_SHARED((R, d), jnp.float32),
            pltpu.VMEM((IDXB, EPW), jnp.int32),
            pltpu.VMEM((IDXB, EPW), jnp.int32),
            pltpu.VMEM((EPW, d), jnp.float32),
            pltpu.SemaphoreType.DMA,
        ],
    )
    sums = sums_call(features, srcp, dstp)

    cnts_call = pl.kernel(
        _cnts_body,
        out_type=jax.ShapeDtypeStruct((NC * R, 128), jnp.float32),
        mesh=mesh,
        scratch_types=[
            pltpu.VMEM_SHARED((R, 128), jnp.float32),
            pltpu.VMEM((IDXB, EPW), jnp.int32),
            pltpu.VMEM((EPW, 128), jnp.float32),
            pltpu.SemaphoreType.DMA,
        ],
    )
    cnts = cnts_call(dstp)

    s0, s1 = sums[:n_nodes], sums[R:R + n_nodes]
    c0, c1 = cnts[:n_nodes], cnts[R:R + n_nodes]

    blk = 400
    grid = n_nodes // blk
    out = pl.pallas_call(
        _combine_body,
        out_shape=jax.ShapeDtypeStruct((n_nodes, d), jnp.float32),
        grid=(grid,),
        in_specs=[
            pl.BlockSpec((blk, d), lambda i: (i, 0)),
            pl.BlockSpec((blk, d), lambda i: (i, 0)),
            pl.BlockSpec((blk, 128), lambda i: (i, 0)),
            pl.BlockSpec((blk, 128), lambda i: (i, 0)),
        ],
        out_specs=pl.BlockSpec((blk, d), lambda i: (i, 0)),
    )(s0, s1, c0, c1)
    return out


# speedup vs baseline: 3.4238x; 1.0936x over previous
import functools

import jax
import jax.numpy as jnp
from jax import lax
from jax.experimental import pallas as pl
from jax.experimental.pallas import tpu as pltpu
from jax.experimental.pallas import tpu_sc as plsc

NC = 2
NS = 16
NW = NC * NS
EPW = 128
IDXB = 8
R_CHUNKS = 80
R = R_CHUNKS * 128


def _zero_rows(rows_v):
    zf = jnp.zeros((16,), jnp.float32)

    @pl.loop(0, EPW)
    def _init(i):
        for k in range(8):
            rows_v[i, pl.ds(k * 16, 16)] = zf


def _sums_body(feat, srcp, dstp, sums_out,
               sh_sums, src_v, dst_v, buf0, buf1, sem0, sem1):
    c = lax.axis_index("c")
    s = lax.axis_index("s")
    w = c * NS + s
    rows_per_w = srcp.shape[0] // NW

    _zero_rows(buf0)

    for k in range(R_CHUNKS // NS):
        chunk = k * NS + s
        pltpu.sync_copy(buf0, sh_sums.at[pl.ds(chunk * 128, 128)])

    pltpu.sync_copy(srcp.at[pl.ds(w * rows_per_w, rows_per_w)], src_v)
    pltpu.sync_copy(dstp.at[pl.ds(w * rows_per_w, rows_per_w)], dst_v)

    plsc.subcore_barrier()

    bufs = [buf0, buf1]
    sems = [sem0, sem1]
    dummy = feat.at[pl.ds(0, 128)]
    pltpu.async_copy(feat.at[src_v.at[0]], buf0, sem0)
    pltpu.async_copy(feat.at[src_v.at[1]], buf1, sem1)

    @pl.loop(0, (rows_per_w - 2) // 2)
    def _main(b):
        for t in range(2):
            r = 2 * b + t
            pltpu.make_async_copy(dummy, bufs[t], sems[t]).wait()
            pltpu.sync_copy(bufs[t], sh_sums.at[dst_v.at[r]], add=True)
            pltpu.async_copy(feat.at[src_v.at[r + 2]], bufs[t], sems[t])

    for t in range(2):
        r = rows_per_w - 2 + t
        pltpu.make_async_copy(dummy, bufs[t], sems[t]).wait()
        pltpu.sync_copy(bufs[t], sh_sums.at[dst_v.at[r]], add=True)

    plsc.subcore_barrier()

    for k in range(R_CHUNKS // NS):
        chunk = k * NS + s
        pltpu.sync_copy(sh_sums.at[pl.ds(chunk * 128, 128)], buf0)
        pltpu.sync_copy(buf0, sums_out.at[pl.ds(c * R + chunk * 128, 128)])


def _cnts_body(dstp, cnts_out, sh_cnts, dst_v, ones_v, sem):
    c = lax.axis_index("c")
    s = lax.axis_index("s")
    w = c * NS + s
    rows_per_w = dstp.shape[0] // NW
    of = jnp.ones((16,), jnp.float32)

    _zero_rows(ones_v)

    for k in range(R_CHUNKS // NS):
        chunk = k * NS + s
        pltpu.sync_copy(ones_v, sh_cnts.at[pl.ds(chunk * 128, 128)])

    @pl.loop(0, EPW)
    def _setones(i):
        for k in range(8):
            ones_v[i, pl.ds(k * 16, 16)] = of

    plsc.subcore_barrier()

    @pl.loop(0, rows_per_w // IDXB)
    def _main(b):
        base = w * rows_per_w + b * IDXB
        pltpu.sync_copy(dstp.at[pl.ds(base, IDXB)], dst_v)
        for jj in range(IDXB):
            pltpu.sync_copy(ones_v, sh_cnts.at[dst_v.at[jj]], add=True)

    plsc.subcore_barrier()

    for k in range(R_CHUNKS // NS):
        chunk = k * NS + s
        pltpu.sync_copy(sh_cnts.at[pl.ds(chunk * 128, 128)], ones_v)
        pltpu.sync_copy(ones_v, cnts_out.at[pl.ds(c * R + chunk * 128, 128)])


def _combine_body(s0, s1, c0, c1, o):
    cnt = c0[:, 0:1] + c1[:, 0:1]
    o[:, :] = (s0[:, :] + s1[:, :]) / jnp.maximum(cnt, 1.0)


def kernel(features, edge_src, edge_dst):
    n_nodes, d = features.shape
    e = edge_src.shape[0]
    rows_per_w = -(-e // (NW * EPW))
    rows_per_w = -(-rows_per_w // IDXB) * IDXB
    e_pad = NW * EPW * rows_per_w
    pad = e_pad - e

    srcp = jnp.concatenate(
        [edge_src, jnp.zeros((pad,), jnp.int32)]).reshape(-1, EPW)
    dstp = jnp.concatenate(
        [edge_dst, jnp.full((pad,), n_nodes, jnp.int32)]).reshape(-1, EPW)

    mesh = plsc.VectorSubcoreMesh(
        core_axis_name="c", subcore_axis_name="s",
        num_cores=NC, num_subcores=NS)

    sums_call = pl.kernel(
        _sums_body,
        out_type=jax.ShapeDtypeStruct((NC * R, d), jnp.float32),
        mesh=mesh,
        scratch_types=[
            pltpu.VMEM_SHARED((R, d), jnp.float32),
            pltpu.VMEM((rows_per_w, EPW), jnp.int32),
            pltpu.VMEM((rows_per_w, EPW), jnp.int32),
            pltpu.VMEM((EPW, d), jnp.float32),
            pltpu.VMEM((EPW, d), jnp.float32),
            pltpu.SemaphoreType.DMA,
            pltpu.SemaphoreType.DMA,
        ],
    )
    sums = sums_call(features, srcp, dstp)

    cnts_call = pl.kernel(
        _cnts_body,
        out_type=jax.ShapeDtypeStruct((NC * R, 128), jnp.float32),
        mesh=mesh,
        scratch_types=[
            pltpu.VMEM_SHARED((R, 128), jnp.float32),
            pltpu.VMEM((IDXB, EPW), jnp.int32),
            pltpu.VMEM((EPW, 128), jnp.float32),
            pltpu.SemaphoreType.DMA,
        ],
    )
    cnts = cnts_call(dstp)

    s0, s1 = sums[:n_nodes], sums[R:R + n_nodes]
    c0, c1 = cnts[:n_nodes], cnts[R:R + n_nodes]

    blk = 400
    grid = n_nodes // blk
    out = pl.pallas_call(
        _combine_body,
        out_shape=jax.ShapeDtypeStruct((n_nodes, d), jnp.float32),
        grid=(grid,),
        in_specs=[
            pl.BlockSpec((blk, d), lambda i: (i, 0)),
            pl.BlockSpec((blk, d), lambda i: (i, 0)),
            pl.BlockSpec((blk, 128), lambda i: (i, 0)),
            pl.BlockSpec((blk, 128), lambda i: (i, 0)),
        ],
        out_specs=pl.BlockSpec((blk, d), lambda i: (i, 0)),
    )(s0, s1, c0, c1)
    return out


# speedup vs baseline: 4.1848x; 1.2223x over previous
import functools

import jax
import jax.numpy as jnp
from jax import lax
from jax.experimental import pallas as pl
from jax.experimental.pallas import tpu as pltpu
from jax.experimental.pallas import tpu_sc as plsc

NC = 2
NS = 16
NW = NC * NS
EPW = 128
IDXB = 8
R_CHUNKS = 80
R = R_CHUNKS * 128


def _zero_rows(rows_v):
    zf = jnp.zeros((16,), jnp.float32)

    @pl.loop(0, EPW)
    def _init(i):
        for k in range(8):
            rows_v[i, pl.ds(k * 16, 16)] = zf


def _sums_body(featb, srcp, dstp, sums_out,
               sh_sums, src_v, dst_v, bufb0, bufb1, buff, sem0, sem1):
    c = lax.axis_index("c")
    s = lax.axis_index("s")
    w = c * NS + s
    rows_per_w = srcp.shape[0] // NW

    _zero_rows(buff)

    for k in range(R_CHUNKS // NS):
        chunk = k * NS + s
        pltpu.sync_copy(buff, sh_sums.at[pl.ds(chunk * 128, 128)])

    pltpu.sync_copy(srcp.at[pl.ds(w * rows_per_w, rows_per_w)], src_v)
    pltpu.sync_copy(dstp.at[pl.ds(w * rows_per_w, rows_per_w)], dst_v)

    plsc.subcore_barrier()

    bufs = [bufb0, bufb1]
    sems = [sem0, sem1]
    dummy = featb.at[pl.ds(0, 128)]
    pltpu.async_copy(featb.at[src_v.at[0]], bufb0, sem0)
    pltpu.async_copy(featb.at[src_v.at[1]], bufb1, sem1)

    hi_mask = jnp.full((16,), 0xFFFF0000, dtype=jnp.uint32)
    sixteen = jnp.full((16,), 16, dtype=jnp.uint32)

    def _round(r, t, start_next):
        pltpu.make_async_copy(dummy, bufs[t], sems[t]).wait()

        @pl.loop(0, EPW)
        def _cvt(i):
            for k in range(4):
                u = bufs[t][i, pl.ds(k * 16, 16)]
                lo = jax.lax.bitcast_convert_type(u << sixteen, jnp.float32)
                hi = jax.lax.bitcast_convert_type(u & hi_mask, jnp.float32)
                buff[i, pl.ds(k * 32, 16)] = lo
                buff[i, pl.ds(k * 32 + 16, 16)] = hi

        if start_next:
            pltpu.async_copy(featb.at[src_v.at[r + 2]], bufs[t], sems[t])
        pltpu.sync_copy(buff, sh_sums.at[dst_v.at[r]], add=True)

    @pl.loop(0, (rows_per_w - 2) // 2)
    def _main(b):
        for t in range(2):
            _round(2 * b + t, t, True)

    for t in range(2):
        _round(rows_per_w - 2 + t, t, False)

    plsc.subcore_barrier()

    for k in range(R_CHUNKS // NS):
        chunk = k * NS + s
        pltpu.sync_copy(sh_sums.at[pl.ds(chunk * 128, 128)], buff)
        pltpu.sync_copy(buff, sums_out.at[pl.ds(c * R + chunk * 128, 128)])


def _cnts_body(dstp, cnts_out, sh_cnts, dst_v, ones_v, sem):
    c = lax.axis_index("c")
    s = lax.axis_index("s")
    w = c * NS + s
    rows_per_w = dstp.shape[0] // NW
    of = jnp.ones((16,), jnp.float32)

    _zero_rows(ones_v)

    for k in range(R_CHUNKS // NS):
        chunk = k * NS + s
        pltpu.sync_copy(ones_v, sh_cnts.at[pl.ds(chunk * 128, 128)])

    @pl.loop(0, EPW)
    def _setones(i):
        for k in range(8):
            ones_v[i, pl.ds(k * 16, 16)] = of

    plsc.subcore_barrier()

    @pl.loop(0, rows_per_w // IDXB)
    def _main(b):
        base = w * rows_per_w + b * IDXB
        pltpu.sync_copy(dstp.at[pl.ds(base, IDXB)], dst_v)
        for jj in range(IDXB):
            pltpu.sync_copy(ones_v, sh_cnts.at[dst_v.at[jj]], add=True)

    plsc.subcore_barrier()

    for k in range(R_CHUNKS // NS):
        chunk = k * NS + s
        pltpu.sync_copy(sh_cnts.at[pl.ds(chunk * 128, 128)], ones_v)
        pltpu.sync_copy(ones_v, cnts_out.at[pl.ds(c * R + chunk * 128, 128)])


def _combine_body(s0, s1, c0, c1, o):
    cnt = c0[:, 0:1] + c1[:, 0:1]
    o[:, :] = (s0[:, :] + s1[:, :]) / jnp.maximum(cnt, 1.0)


def kernel(features, edge_src, edge_dst):
    n_nodes, d = features.shape
    e = edge_src.shape[0]
    rows_per_w = -(-e // (NW * EPW))
    rows_per_w = -(-rows_per_w // IDXB) * IDXB
    e_pad = NW * EPW * rows_per_w
    pad = e_pad - e

    srcp = jnp.concatenate(
        [edge_src, jnp.zeros((pad,), jnp.int32)]).reshape(-1, EPW)
    dstp = jnp.concatenate(
        [edge_dst, jnp.full((pad,), n_nodes, jnp.int32)]).reshape(-1, EPW)

    import numpy as np
    perm = np.empty((d,), dtype=np.int32)
    for kb in range(d // 32):
        for t in range(16):
            perm[kb * 32 + 2 * t] = kb * 32 + t
            perm[kb * 32 + 2 * t + 1] = kb * 32 + 16 + t
    featb = features[:, perm].astype(jnp.bfloat16)
    featu = jax.lax.bitcast_convert_type(
        featb.reshape(n_nodes, d // 2, 2), jnp.uint32)

    mesh = plsc.VectorSubcoreMesh(
        core_axis_name="c", subcore_axis_name="s",
        num_cores=NC, num_subcores=NS)

    sums_call = pl.kernel(
        _sums_body,
        out_type=jax.ShapeDtypeStruct((NC * R, d), jnp.float32),
        mesh=mesh,
        scratch_types=[
            pltpu.VMEM_SHARED((R, d), jnp.float32),
            pltpu.VMEM((rows_per_w, EPW), jnp.int32),
            pltpu.VMEM((rows_per_w, EPW), jnp.int32),
            pltpu.VMEM((EPW, d // 2), jnp.uint32),
            pltpu.VMEM((EPW, d // 2), jnp.uint32),
            pltpu.VMEM((EPW, d), jnp.float32),
            pltpu.SemaphoreType.DMA,
            pltpu.SemaphoreType.DMA,
        ],
        compiler_params=pltpu.CompilerParams(use_tc_tiling_on_sc=False),
    )
    sums = sums_call(featu, srcp, dstp)

    cnts_call = pl.kernel(
        _cnts_body,
        out_type=jax.ShapeDtypeStruct((NC * R, 128), jnp.float32),
        mesh=mesh,
        scratch_types=[
            pltpu.VMEM_SHARED((R, 128), jnp.float32),
            pltpu.VMEM((IDXB, EPW), jnp.int32),
            pltpu.VMEM((EPW, 128), jnp.float32),
            pltpu.SemaphoreType.DMA,
        ],
    )
    cnts = cnts_call(dstp)

    s0, s1 = sums[:n_nodes], sums[R:R + n_nodes]
    c0, c1 = cnts[:n_nodes], cnts[R:R + n_nodes]

    blk = 400
    grid = n_nodes // blk
    out = pl.pallas_call(
        _combine_body,
        out_shape=jax.ShapeDtypeStruct((n_nodes, d), jnp.float32),
        grid=(grid,),
        in_specs=[
            pl.BlockSpec((blk, d), lambda i: (i, 0)),
            pl.BlockSpec((blk, d), lambda i: (i, 0)),
            pl.BlockSpec((blk, 128), lambda i: (i, 0)),
            pl.BlockSpec((blk, 128), lambda i: (i, 0)),
        ],
        out_specs=pl.BlockSpec((blk, d), lambda i: (i, 0)),
    )(s0, s1, c0, c1)
    return out


# speedup vs baseline: 5.1863x; 1.2393x over previous
import functools

import jax
import jax.numpy as jnp
from jax import lax
from jax.experimental import pallas as pl
from jax.experimental.pallas import tpu as pltpu
from jax.experimental.pallas import tpu_sc as plsc

NC = 2
NS = 16
NW = NC * NS
EPW = 128
IDXB = 8
R_CHUNKS = 80
R = R_CHUNKS * 128


def _zero_rows(rows_v):
    zf = jnp.zeros((16,), jnp.float32)

    @pl.loop(0, EPW)
    def _init(i):
        for k in range(8):
            rows_v[i, pl.ds(k * 16, 16)] = zf


def _sums_body(featb, srcp, dstp, sums_out,
               sh_sums, src_v, dst_v, bufb0, bufb1, buff, sem0, sem1):
    c = lax.axis_index("c")
    s = lax.axis_index("s")
    w = c * NS + s
    rows_per_w = srcp.shape[0] // NW

    _zero_rows(buff)

    for k in range(R_CHUNKS // NS):
        chunk = k * NS + s
        pltpu.sync_copy(buff, sh_sums.at[pl.ds(chunk * 128, 128)])

    pltpu.sync_copy(srcp.at[pl.ds(w * rows_per_w, rows_per_w)], src_v)
    pltpu.sync_copy(dstp.at[pl.ds(w * rows_per_w, rows_per_w)], dst_v)

    plsc.subcore_barrier()

    bufs = [bufb0, bufb1]
    sems = [sem0, sem1]
    dummy = featb.at[pl.ds(0, 128)]
    pltpu.async_copy(featb.at[src_v.at[0]], bufb0, sem0)
    pltpu.async_copy(featb.at[src_v.at[1]], bufb1, sem1)

    hi_mask = jnp.full((16,), 0xFFFF0000, dtype=jnp.uint32)
    sixteen = jnp.full((16,), 16, dtype=jnp.uint32)

    def _round(r, t, start_next):
        pltpu.make_async_copy(dummy, bufs[t], sems[t]).wait()

        @pl.loop(0, EPW)
        def _cvt(i):
            for k in range(4):
                u = bufs[t][i, pl.ds(k * 16, 16)]
                lo = jax.lax.bitcast_convert_type(u << sixteen, jnp.float32)
                hi = jax.lax.bitcast_convert_type(u & hi_mask, jnp.float32)
                buff[i, pl.ds(k * 32, 16)] = lo
                buff[i, pl.ds(k * 32 + 16, 16)] = hi

        if start_next:
            pltpu.async_copy(featb.at[src_v.at[r + 2]], bufs[t], sems[t])
        pltpu.sync_copy(buff, sh_sums.at[dst_v.at[r]], add=True)

    @pl.loop(0, (rows_per_w - 2) // 2)
    def _main(b):
        for t in range(2):
            _round(2 * b + t, t, True)

    for t in range(2):
        _round(rows_per_w - 2 + t, t, False)

    plsc.subcore_barrier()

    for k in range(R_CHUNKS // NS):
        chunk = k * NS + s
        pltpu.sync_copy(sh_sums.at[pl.ds(chunk * 128, 128)], buff)
        pltpu.sync_copy(buff, sums_out.at[pl.ds(c * R + chunk * 128, 128)])


def _cnts_body(dstp, cnts_out, sh_cnts, dst_v, ones_v, sem):
    c = lax.axis_index("c")
    s = lax.axis_index("s")
    w = c * NS + s
    rows_per_w = dstp.shape[0] // NW
    zf = jnp.zeros((16,), jnp.float32)
    of = jnp.ones((16,), jnp.float32)

    @pl.loop(0, EPW)
    def _init(i):
        ones_v[i, pl.ds(0, 16)] = zf

    for k in range(R_CHUNKS // NS):
        chunk = k * NS + s
        pltpu.sync_copy(ones_v, sh_cnts.at[pl.ds(chunk * 128, 128)])

    @pl.loop(0, EPW)
    def _setones(i):
        ones_v[i, pl.ds(0, 16)] = of

    plsc.subcore_barrier()

    @pl.loop(0, rows_per_w // IDXB)
    def _main(b):
        base = w * rows_per_w + b * IDXB
        pltpu.sync_copy(dstp.at[pl.ds(base, IDXB)], dst_v)
        for jj in range(IDXB):
            pltpu.sync_copy(ones_v, sh_cnts.at[dst_v.at[jj]], add=True)

    plsc.subcore_barrier()

    for k in range(R_CHUNKS // NS):
        chunk = k * NS + s
        pltpu.sync_copy(sh_cnts.at[pl.ds(chunk * 128, 128)], ones_v)
        pltpu.sync_copy(ones_v, cnts_out.at[pl.ds(c * R + chunk * 128, 128)])


def _combine_body(s0, s1, c0, c1, o):
    cnt = c0[:, 0:1] + c1[:, 0:1]
    o[:, :] = (s0[:, :] + s1[:, :]) / jnp.maximum(cnt, 1.0)


def kernel(features, edge_src, edge_dst):
    n_nodes, d = features.shape
    e = edge_src.shape[0]
    rows_per_w = -(-e // (NW * EPW))
    rows_per_w = -(-rows_per_w // IDXB) * IDXB
    e_pad = NW * EPW * rows_per_w
    pad = e_pad - e

    srcp = jnp.concatenate(
        [edge_src, jnp.zeros((pad,), jnp.int32)]).reshape(-1, EPW)
    dstp = jnp.concatenate(
        [edge_dst, jnp.full((pad,), n_nodes, jnp.int32)]).reshape(-1, EPW)

    bits = jax.lax.bitcast_convert_type(features, jnp.uint32)
    r = bits + (jnp.uint32(0x7FFF) + ((bits >> 16) & jnp.uint32(1)))
    r4 = r.reshape(n_nodes, d // 32, 2, 16)
    featu = ((r4[:, :, 1, :] & jnp.uint32(0xFFFF0000))
             | (r4[:, :, 0, :] >> 16)).reshape(n_nodes, d // 2)

    mesh = plsc.VectorSubcoreMesh(
        core_axis_name="c", subcore_axis_name="s",
        num_cores=NC, num_subcores=NS)

    sums_call = pl.kernel(
        _sums_body,
        out_type=jax.ShapeDtypeStruct((NC * R, d), jnp.float32),
        mesh=mesh,
        scratch_types=[
            pltpu.VMEM_SHARED((R, d), jnp.float32),
            pltpu.VMEM((rows_per_w, EPW), jnp.int32),
            pltpu.VMEM((rows_per_w, EPW), jnp.int32),
            pltpu.VMEM((EPW, d // 2), jnp.uint32),
            pltpu.VMEM((EPW, d // 2), jnp.uint32),
            pltpu.VMEM((EPW, d), jnp.float32),
            pltpu.SemaphoreType.DMA,
            pltpu.SemaphoreType.DMA,
        ],
        compiler_params=pltpu.CompilerParams(use_tc_tiling_on_sc=False),
    )
    sums = sums_call(featu, srcp, dstp)

    cnts_call = pl.kernel(
        _cnts_body,
        out_type=jax.ShapeDtypeStruct((NC * R, 16), jnp.float32),
        mesh=mesh,
        scratch_types=[
            pltpu.VMEM_SHARED((R, 16), jnp.float32),
            pltpu.VMEM((IDXB, EPW), jnp.int32),
            pltpu.VMEM((EPW, 16), jnp.float32),
            pltpu.SemaphoreType.DMA,
        ],
        compiler_params=pltpu.CompilerParams(use_tc_tiling_on_sc=False),
    )
    cnts = cnts_call(dstp)

    blk = 128
    grid = -(-n_nodes // blk)
    out = pl.pallas_call(
        _combine_body,
        out_shape=jax.ShapeDtypeStruct((n_nodes, d), jnp.float32),
        grid=(grid,),
        in_specs=[
            pl.BlockSpec((blk, d), lambda i: (i, 0)),
            pl.BlockSpec((blk, d), lambda i: (i + R_CHUNKS, 0)),
            pl.BlockSpec((blk, 16), lambda i: (i, 0)),
            pl.BlockSpec((blk, 16), lambda i: (i + R_CHUNKS, 0)),
        ],
        out_specs=pl.BlockSpec((blk, d), lambda i: (i, 0)),
    )(sums, sums, cnts, cnts)
    return out


# speedup vs baseline: 6.2388x; 1.2029x over previous
import functools

import jax
import jax.numpy as jnp
from jax import lax
from jax.experimental import pallas as pl
from jax.experimental.pallas import tpu as pltpu
from jax.experimental.pallas import tpu_sc as plsc

NC = 2
NS = 16
NW = NC * NS
EPW = 128
IDXB = 8
R_CHUNKS = 80
R = R_CHUNKS * 128


def _zero_rows(rows_v):
    zf = jnp.zeros((16,), jnp.float32)

    @pl.loop(0, EPW)
    def _init(i):
        for k in range(8):
            rows_v[i, pl.ds(k * 16, 16)] = zf


def _sums_body(featb, srcp, dstp, sums_out,
               sh_sums, src_v, dst_v, bufb0, bufb1, buff, sem0, sem1):
    c = lax.axis_index("c")
    s = lax.axis_index("s")
    w = c * NS + s
    rows_per_w = srcp.shape[0] // NW

    _zero_rows(buff)

    for k in range(R_CHUNKS // NS):
        chunk = k * NS + s
        pltpu.sync_copy(buff, sh_sums.at[pl.ds(chunk * 128, 128)])

    pltpu.sync_copy(srcp.at[pl.ds(w * rows_per_w, rows_per_w)], src_v)
    pltpu.sync_copy(dstp.at[pl.ds(w * rows_per_w, rows_per_w)], dst_v)

    plsc.subcore_barrier()

    bufs = [bufb0, bufb1]
    sems = [sem0, sem1]
    dummy = featb.at[pl.ds(0, 128)]
    pltpu.async_copy(featb.at[src_v.at[0]], bufb0, sem0)
    pltpu.async_copy(featb.at[src_v.at[1]], bufb1, sem1)

    hi_mask = jnp.full((16,), 0xFFFF0000, dtype=jnp.uint32)
    sixteen = jnp.full((16,), 16, dtype=jnp.uint32)

    def _round(r, t, start_next):
        pltpu.make_async_copy(dummy, bufs[t], sems[t]).wait()

        @pl.loop(0, EPW)
        def _cvt(i):
            for k in range(4):
                u = bufs[t][i, pl.ds(k * 16, 16)]
                lo = jax.lax.bitcast_convert_type(u << sixteen, jnp.float32)
                hi = jax.lax.bitcast_convert_type(u & hi_mask, jnp.float32)
                buff[i, pl.ds(k * 32, 16)] = lo
                buff[i, pl.ds(k * 32 + 16, 16)] = hi

        if start_next:
            pltpu.async_copy(featb.at[src_v.at[r + 2]], bufs[t], sems[t])
        pltpu.sync_copy(buff, sh_sums.at[dst_v.at[r]], add=True)

    @pl.loop(0, (rows_per_w - 2) // 2)
    def _main(b):
        for t in range(2):
            _round(2 * b + t, t, True)

    for t in range(2):
        _round(rows_per_w - 2 + t, t, False)

    plsc.subcore_barrier()

    for k in range(R_CHUNKS // NS):
        chunk = k * NS + s
        pltpu.sync_copy(sh_sums.at[pl.ds(chunk * 128, 128)], buff)
        pltpu.sync_copy(buff, sums_out.at[pl.ds(c * R + chunk * 128, 128)])


def _cnts_body(dstp, cnts_out, sh_cnts, dst_v, ones_v, sem):
    c = lax.axis_index("c")
    s = lax.axis_index("s")
    w = c * NS + s
    rows_per_w = dstp.shape[0] // NW
    zf = jnp.zeros((16,), jnp.float32)
    of = jnp.ones((16,), jnp.float32)

    @pl.loop(0, EPW)
    def _init(i):
        ones_v[i, pl.ds(0, 16)] = zf

    for k in range(R_CHUNKS // NS):
        chunk = k * NS + s
        pltpu.sync_copy(ones_v, sh_cnts.at[pl.ds(chunk * 128, 128)])

    @pl.loop(0, EPW)
    def _setones(i):
        ones_v[i, pl.ds(0, 16)] = of

    plsc.subcore_barrier()

    @pl.loop(0, rows_per_w // IDXB)
    def _main(b):
        base = w * rows_per_w + b * IDXB
        pltpu.sync_copy(dstp.at[pl.ds(base, IDXB)], dst_v)
        for jj in range(IDXB):
            pltpu.sync_copy(ones_v, sh_cnts.at[dst_v.at[jj]], add=True)

    plsc.subcore_barrier()

    for k in range(R_CHUNKS // NS):
        chunk = k * NS + s
        pltpu.sync_copy(sh_cnts.at[pl.ds(chunk * 128, 128)], ones_v)
        pltpu.sync_copy(ones_v, cnts_out.at[pl.ds(c * R + chunk * 128, 128)])


def _combine_body(s0, s1, c0, c1, o):
    cnt = c0[:, 0:1] + c1[:, 0:1]
    o[:, :] = (s0[:, :] + s1[:, :]) / jnp.maximum(cnt, 1.0)


def kernel(features, edge_src, edge_dst):
    n_nodes, d = features.shape
    e = edge_src.shape[0]
    rows_per_w = -(-e // (NW * EPW))
    rows_per_w = -(-rows_per_w // IDXB) * IDXB
    e_pad = NW * EPW * rows_per_w
    pad = e_pad - e

    srcp = jnp.concatenate(
        [edge_src, jnp.zeros((pad,), jnp.int32)]).reshape(-1, EPW)
    dstp = jnp.concatenate(
        [edge_dst, jnp.full((pad,), n_nodes, jnp.int32)]).reshape(-1, EPW)

    bits = jax.lax.bitcast_convert_type(features, jnp.uint32)
    r = bits + (jnp.uint32(0x7FFF) + ((bits >> 16) & jnp.uint32(1)))
    r4 = r.reshape(n_nodes, d // 32, 2, 16)
    featu = ((r4[:, :, 1, :] & jnp.uint32(0xFFFF0000))
             | (r4[:, :, 0, :] >> 16)).reshape(n_nodes, d // 2)

    mesh = plsc.VectorSubcoreMesh(
        core_axis_name="c", subcore_axis_name="s",
        num_cores=NC, num_subcores=NS)

    sums_call = pl.kernel(
        _sums_body,
        out_type=jax.ShapeDtypeStruct((NC * R, d), jnp.float32),
        mesh=mesh,
        scratch_types=[
            pltpu.VMEM_SHARED((R, d), jnp.float32),
            pltpu.VMEM((rows_per_w, EPW), jnp.int32),
            pltpu.VMEM((rows_per_w, EPW), jnp.int32),
            pltpu.VMEM((EPW, d // 2), jnp.uint32),
            pltpu.VMEM((EPW, d // 2), jnp.uint32),
            pltpu.VMEM((EPW, d), jnp.float32),
            pltpu.SemaphoreType.DMA,
            pltpu.SemaphoreType.DMA,
        ],
        compiler_params=pltpu.CompilerParams(use_tc_tiling_on_sc=False),
    )
    sums = sums_call(featu, srcp, dstp)

    cnts_call = pl.kernel(
        _cnts_body,
        out_type=jax.ShapeDtypeStruct((NC * R, 16), jnp.float32),
        mesh=mesh,
        scratch_types=[
            pltpu.VMEM_SHARED((R, 16), jnp.float32),
            pltpu.VMEM((IDXB, EPW), jnp.int32),
            pltpu.VMEM((EPW, 16), jnp.float32),
            pltpu.SemaphoreType.DMA,
        ],
        compiler_params=pltpu.CompilerParams(use_tc_tiling_on_sc=False),
    )
    cnts = cnts_call(dstp)

    blk = 1024
    off = R // blk
    grid = -(-n_nodes // blk)
    out = pl.pallas_call(
        _combine_body,
        out_shape=jax.ShapeDtypeStruct((n_nodes, d), jnp.float32),
        grid=(grid,),
        in_specs=[
            pl.BlockSpec((blk, d), lambda i: (i, 0)),
            pl.BlockSpec((blk, d), lambda i, off=off: (i + off, 0)),
            pl.BlockSpec((blk, 16), lambda i: (i, 0)),
            pl.BlockSpec((blk, 16), lambda i, off=off: (i + off, 0)),
        ],
        out_specs=pl.BlockSpec((blk, d), lambda i: (i, 0)),
    )(sums, sums, cnts, cnts)
    return out
